# baseline (speedup 1.0000x reference)
import functools

import jax
import jax.numpy as jnp
from jax.experimental import pallas as pl
from jax.experimental.pallas import tpu as pltpu

_HALF = 64
_CHUNK = 256
_TILE_B = 8192


def _ceil_to(x, m):
    return ((x + m - 1) // m) * m


def _embed_tile_kernel(x_ref, tbl_ref, out_ref):
    tb, num_cards = x_ref.shape
    tbl = tbl_ref[...]
    iota = jax.lax.broadcasted_iota(jnp.int32, (_HALF, _CHUNK), 0)

    def body(c, carry):
        r0 = pl.multiple_of(c * _CHUNK, _CHUNK)
        xv = x_ref[pl.ds(r0, _CHUNK), :].T
        lo = (xv[0:1, :] == iota).astype(jnp.float32)
        hi = (xv[1:2, :] == iota).astype(jnp.float32)
        for j in range(2, num_cards, 2):
            lo = lo + (xv[j:j + 1, :] == iota).astype(jnp.float32)
            hi = hi + (xv[j + 1:j + 2, :] == iota).astype(jnp.float32)
        counts_t = jnp.concatenate([lo, hi], axis=0)
        out_ref[pl.ds(r0, _CHUNK), :] = jax.lax.dot_general(
            counts_t, tbl, (((0,), (0,)), ((), ())),
            preferred_element_type=jnp.float32)
        return carry

    jax.lax.fori_loop(0, tb // _CHUNK, body, None, unroll=8)


@jax.jit
def kernel(indices, card_w, rank_w, suit_w):
    B, num_cards = indices.shape
    vocab, dim = card_w.shape
    dp = _ceil_to(dim, 128)

    cards = jnp.arange(vocab, dtype=jnp.int32)
    tbl = card_w + rank_w[cards // 4] + suit_w[cards % 4]
    tbl = jnp.pad(tbl, ((0, _HALF - vocab), (0, dp - dim)))
    tbl2 = jnp.concatenate([tbl, tbl], axis=0)

    tb = min(_TILE_B, _ceil_to(B, _CHUNK))
    bp = _ceil_to(B, tb)
    x = indices.astype(jnp.int32)
    if bp != B:
        x = jnp.pad(x, ((0, bp - B), (0, 0)), constant_values=-1)
    if num_cards % 2:
        x = jnp.pad(x, ((0, 0), (0, 1)), constant_values=-1)
        num_cards += 1

    out = pl.pallas_call(
        _embed_tile_kernel,
        out_shape=jax.ShapeDtypeStruct((bp, dp), jnp.float32),
        grid=(bp // tb,),
        in_specs=[
            pl.BlockSpec((tb, num_cards), lambda i: (i, 0)),
            pl.BlockSpec((2 * _HALF, dp), lambda i: (0, 0)),
        ],
        out_specs=pl.BlockSpec((tb, dp), lambda i: (i, 0)),
        compiler_params=pltpu.CompilerParams(
            dimension_semantics=("parallel",),
            vmem_limit_bytes=48 << 20),
    )(x, tbl2)

    if bp != B or dp != dim:
        out = out[:B, :dim]
    return out


# speedup vs baseline: 1.6431x; 1.6431x over previous
import functools

import jax
import jax.numpy as jnp
from jax.experimental import pallas as pl
from jax.experimental.pallas import tpu as pltpu

_HALF = 64
_CHUNK = 256
_TILE_B = 8192


def _ceil_to(x, m):
    return ((x + m - 1) // m) * m


def _embed_tile_kernel(xt_ref, tbl_ref, out_ref):
    num_cards, tb = xt_ref.shape
    tbl = tbl_ref[...]
    iota = jax.lax.broadcasted_iota(jnp.int32, (_HALF, _CHUNK), 0)

    def body(c, carry):
        r0 = pl.multiple_of(c * _CHUNK, _CHUNK)
        xv = xt_ref[:, pl.ds(r0, _CHUNK)]
        lo = (xv[0:1, :] == iota).astype(jnp.float32)
        hi = (xv[1:2, :] == iota).astype(jnp.float32)
        for j in range(2, num_cards, 2):
            lo = lo + (xv[j:j + 1, :] == iota).astype(jnp.float32)
            hi = hi + (xv[j + 1:j + 2, :] == iota).astype(jnp.float32)
        counts_t = jnp.concatenate([lo, hi], axis=0)
        out_ref[pl.ds(r0, _CHUNK), :] = jax.lax.dot_general(
            counts_t, tbl, (((0,), (0,)), ((), ())),
            preferred_element_type=jnp.float32)
        return carry

    jax.lax.fori_loop(0, tb // _CHUNK, body, None, unroll=8)


@jax.jit
def kernel(indices, card_w, rank_w, suit_w):
    B, num_cards = indices.shape
    vocab, dim = card_w.shape
    dp = _ceil_to(dim, 128)

    cards = jnp.arange(vocab, dtype=jnp.int32)
    tbl = card_w + rank_w[cards // 4] + suit_w[cards % 4]
    tbl = jnp.pad(tbl, ((0, _HALF - vocab), (0, dp - dim)))
    tbl2 = jnp.concatenate([tbl, tbl], axis=0)

    tb = min(_TILE_B, _ceil_to(B, _CHUNK))
    bp = _ceil_to(B, tb)
    x = indices.astype(jnp.int32)
    if bp != B:
        x = jnp.pad(x, ((0, bp - B), (0, 0)), constant_values=-1)
    if num_cards % 2:
        x = jnp.pad(x, ((0, 0), (0, 1)), constant_values=-1)
        num_cards += 1
    xt = x.T

    out = pl.pallas_call(
        _embed_tile_kernel,
        out_shape=jax.ShapeDtypeStruct((bp, dp), jnp.float32),
        grid=(bp // tb,),
        in_specs=[
            pl.BlockSpec((num_cards, tb), lambda i: (0, i)),
            pl.BlockSpec((2 * _HALF, dp), lambda i: (0, 0)),
        ],
        out_specs=pl.BlockSpec((tb, dp), lambda i: (i, 0)),
        compiler_params=pltpu.CompilerParams(
            dimension_semantics=("parallel",),
            vmem_limit_bytes=48 << 20),
    )(xt, tbl2)

    if bp != B or dp != dim:
        out = out[:B, :dim]
    return out
